# Initial kernel scaffold; baseline (speedup 1.0000x reference)
import jax
import jax.numpy as jnp
from jax.experimental import pallas as pl


def kernel(x, edge_index, W_l, W_r, att):
    raise NotImplementedError("write your pallas kernel here")


# speedup vs baseline: 1.6126x; 1.6126x over previous
import jax
import jax.numpy as jnp
from jax.experimental import pallas as pl

N = 10000
E = 320000
D_IN = 128
D = 32
NEG = 0.2


def _matmul_body(x_ref, wl_ref, wr_ref, xl_ref, xr_ref):
    xv = x_ref[...]
    xl_ref[...] = jnp.dot(xv, wl_ref[...], preferred_element_type=jnp.float32)
    xr_ref[...] = jnp.dot(xv, wr_ref[...], preferred_element_type=jnp.float32)


def _final_body(a_ref, d_ref, o_ref):
    o_ref[...] = a_ref[...] / (d_ref[...] + 1e-16)


def kernel(x, edge_index, W_l, W_r, att):
    xl, xr = pl.pallas_call(
        _matmul_body,
        out_shape=[jax.ShapeDtypeStruct((N, D), jnp.float32)] * 2,
    )(x.astype(jnp.float32), W_l.astype(jnp.float32),
      W_r.astype(jnp.float32))

    src = edge_index[0]
    dst = edge_index[1]
    att_v = att.reshape(D).astype(jnp.float32)

    xj = xl[src]
    xi = xr[dst]
    e = xj + xi
    e = jnp.maximum(e, NEG * e)
    alpha = e @ att_v
    amax = jax.ops.segment_max(alpha, dst, num_segments=N)
    amax = jnp.where(jnp.isfinite(amax), amax, 0.0)
    ex = jnp.exp(alpha - amax[dst])
    denom = jax.ops.segment_sum(ex, dst, num_segments=N)
    num = jax.ops.segment_sum(ex[:, None] * xj, dst, num_segments=N)

    out = pl.pallas_call(
        _final_body,
        out_shape=jax.ShapeDtypeStruct((N, D), jnp.float32),
    )(num, denom[:, None])
    return out
